# baseline (speedup 1.0000x reference)
import dataclasses
import functools

import jax
import jax.numpy as jnp
from jax import lax
from jax.experimental import pallas as pl
from jax.experimental.pallas import tpu as pltpu
from jax.experimental.pallas import tpu_sc as plsc

N_NODES = 10000
N_EDGES = 320000
EPAD = 327680
N_GRAPHS = 1000
GPAD = 1024
NPAD = 10240
CH = 128
N_PAIRS = 4096

_HIGH = jax.lax.Precision.HIGHEST


@functools.cache
def _mesh():
    return plsc.VectorSubcoreMesh(core_axis_name="c", subcore_axis_name="s")


def _sc_compiler_params():
    cp = pltpu.CompilerParams()
    if "needs_layout_passes" in pltpu.CompilerParams.__dataclass_fields__:
        cp = dataclasses.replace(cp, needs_layout_passes=False)
    return cp


def _sc_untiled_params():
    cp = pltpu.CompilerParams()
    if "use_tc_tiling_on_sc" in pltpu.CompilerParams.__dataclass_fields__:
        cp = dataclasses.replace(cp, use_tc_tiling_on_sc=False)
    return cp


def _dot(a, b):
    return jnp.dot(a, b, preferred_element_type=jnp.float32)


@functools.cache
def _sc_hist():
    return pl.kernel(
        _sc_hist_body,
        out_type=jax.ShapeDtypeStruct((32, NPAD), jnp.float32),
        mesh=_mesh(),
        scratch_types=[
            pltpu.VMEM((EPAD // 32,), jnp.int32),
            pltpu.VMEM((NPAD,), jnp.float32),
        ],
        compiler_params=_sc_compiler_params(),
    )


def _sc_hist_body(dst_hbm, z1d_hbm, out_hbm, idx_v, hist_v):
    cid = lax.axis_index("c")
    sid = lax.axis_index("s")
    w = cid * 16 + sid
    pltpu.sync_copy(z1d_hbm, hist_v)
    pltpu.sync_copy(dst_hbm.at[pl.ds(w * (EPAD // 32), EPAD // 32)], idx_v)
    ones16 = jnp.ones((16,), jnp.float32)

    def body(i, carry):
        for k in range(16):
            ids = idx_v[pl.ds(i * 256 + k * 16, 16)]
            plsc.addupdate_scatter(hist_v, [ids], ones16)
        return carry

    lax.fori_loop(0, EPAD // 32 // 256, body, 0)
    pltpu.sync_copy(hist_v, out_hbm.at[w])


HCH = CH // 2
NCHUNK = EPAD // 16 // 128


@functools.cache
def _sc_edge_pass():
    return pl.kernel(
        _sc_edge_pass_body,
        out_type=jax.ShapeDtypeStruct((2, NPAD, HCH), jnp.float32),
        mesh=_mesh(),
        scratch_types=[
            pltpu.VMEM((NCHUNK, 128), jnp.int32),
            pltpu.VMEM((NCHUNK, 128), jnp.int32),
            pltpu.VMEM((4, 128, HCH), jnp.float32),
            pltpu.VMEM_SHARED((NPAD, HCH), jnp.float32),
            pltpu.SemaphoreType.DMA((4,)),
        ],
        compiler_params=_sc_untiled_params(),
    )


def _sc_edge_pass_body(table_hbm, srclo_hbm, srchi_hbm, dst_hbm, zeros_hbm,
                       out_hbm, src_v, dst_v, rows_v, acc_sh, sem):
    cid = lax.axis_index("c")
    sid = lax.axis_index("s")
    r0 = sid * (NPAD // 16)
    pltpu.sync_copy(zeros_hbm.at[pl.ds(r0, NPAD // 16)], acc_sh.at[pl.ds(r0, NPAD // 16)])

    @pl.when(cid == 0)
    def _():
        pltpu.sync_copy(srclo_hbm.at[pl.ds(sid * NCHUNK, NCHUNK)], src_v)

    @pl.when(cid == 1)
    def _():
        pltpu.sync_copy(srchi_hbm.at[pl.ds(sid * NCHUNK, NCHUNK)], src_v)

    pltpu.sync_copy(dst_hbm.at[pl.ds(sid * NCHUNK, NCHUNK)], dst_v)
    plsc.subcore_barrier()

    def fetch(i, b):
        pltpu.async_copy(table_hbm.at[src_v.at[i]], rows_v.at[b], sem.at[b])

    for i in range(3):
        fetch(i, i)

    def body(i, carry):
        b = lax.rem(i, 4)

        @pl.when(i + 3 < NCHUNK)
        def _():
            fetch(i + 3, lax.rem(i + 3, 4))

        pltpu.make_async_copy(table_hbm.at[src_v.at[i]], rows_v.at[b], sem.at[b]).wait()
        pltpu.sync_copy(rows_v.at[b], acc_sh.at[dst_v.at[i]], add=True)
        return carry

    lax.fori_loop(0, NCHUNK, body, 0)
    plsc.subcore_barrier()
    pltpu.sync_copy(acc_sh.at[pl.ds(r0, NPAD // 16)], out_hbm.at[cid, pl.ds(r0, NPAD // 16)])


@functools.cache
def _sc_pool():
    return pl.kernel(
        _sc_pool_body,
        out_type=[
            jax.ShapeDtypeStruct((2, GPAD, CH), jnp.float32),
            jax.ShapeDtypeStruct((32, GPAD), jnp.float32),
            jax.ShapeDtypeStruct((32, GPAD), jnp.float32),
        ],
        mesh=_mesh(),
        scratch_types=[
            pltpu.VMEM((NPAD // 32,), jnp.int32),
            pltpu.VMEM((8, 40), jnp.int32),
            pltpu.VMEM((256,), jnp.int32),
            pltpu.VMEM((NPAD // 32, CH), jnp.float32),
            pltpu.VMEM((GPAD,), jnp.float32),
            pltpu.VMEM((GPAD,), jnp.float32),
            pltpu.VMEM_SHARED((GPAD, CH), jnp.float32),
        ],
        compiler_params=_sc_compiler_params(),
    )


def _sc_pool_body(h2_hbm, batch_hbm, batch2_hbm, ddb_hbm, zrow_hbm, z1g_hbm,
                  sums_o, cnt_o, pres_o,
                  bidx_v, bidx2_v, didx_v, rows_v, cnt_v, pres_v,
                  sums_sh):
    cid = lax.axis_index("c")
    sid = lax.axis_index("s")
    w = cid * 16 + sid
    g0 = sid * (GPAD // 16)
    gs = GPAD // 16
    R = NPAD // 32
    pltpu.sync_copy(zrow_hbm.at[pl.ds(g0, gs)], sums_sh.at[pl.ds(g0, gs)])
    pltpu.sync_copy(z1g_hbm, cnt_v)
    pltpu.sync_copy(z1g_hbm, pres_v)
    pltpu.sync_copy(batch_hbm.at[pl.ds(w * R, R)], bidx_v)
    pltpu.sync_copy(batch2_hbm.at[pl.ds(w * 8, 8)], bidx2_v)
    pltpu.sync_copy(ddb_hbm.at[pl.ds(w * 256, 256)], didx_v)
    pltpu.sync_copy(h2_hbm.at[pl.ds(w * R, R)], rows_v)
    plsc.subcore_barrier()

    def nbody(c, carry):
        pltpu.sync_copy(rows_v.at[pl.ds(c * 40, 40)], sums_sh.at[bidx2_v.at[c]], add=True)
        return carry

    lax.fori_loop(0, 8, nbody, 0)

    ones16 = jnp.ones((16,), jnp.float32)
    base = w * R

    def cbody(k, carry):
        ids = bidx_v[pl.ds(k * 16, 16)]
        gidx = base + k * 16 + lax.iota(jnp.int32, 16)
        vals = jnp.where(gidx < N_NODES, 1.0, 0.0)
        plsc.addupdate_scatter(cnt_v, [ids], vals)
        return carry

    lax.fori_loop(0, R // 16, cbody, 0)

    def pbody(c, carry):
        for k in range(8):
            ids = didx_v[pl.ds(c * 128 + k * 16, 16)]
            plsc.addupdate_scatter(pres_v, [ids], ones16)
        return carry

    lax.fori_loop(0, 2, pbody, 0)
    plsc.subcore_barrier()
    pltpu.sync_copy(sums_sh.at[pl.ds(g0, gs)], sums_o.at[cid, pl.ds(g0, gs)])
    pltpu.sync_copy(cnt_v, cnt_o.at[w])
    pltpu.sync_copy(pres_v, pres_o.at[w])


@functools.cache
def _sc_pair_gather():
    return pl.kernel(
        _sc_pair_gather_body,
        out_type=jax.ShapeDtypeStruct((2 * N_PAIRS, CH), jnp.float32),
        mesh=_mesh(),
        scratch_types=[
            pltpu.VMEM((GPAD,), jnp.int32),
            pltpu.VMEM((128,), jnp.int32),
            pltpu.VMEM((128,), jnp.int32),
            pltpu.VMEM((128, CH), jnp.float32),
            pltpu.SemaphoreType.DMA,
        ],
        compiler_params=_sc_compiler_params(),
    )


def _sc_pair_gather_body(rank_hbm, ddb_hbm, ge_hbm, out_hbm,
                         rank_v, idx_v, q_v, rows_v, sem):
    cid = lax.axis_index("c")
    sid = lax.axis_index("s")
    w = cid * 16 + sid
    pltpu.sync_copy(rank_hbm, rank_v)

    def body(c, carry):
        base = w * 256 + c * 128
        pltpu.sync_copy(ddb_hbm.at[pl.ds(base, 128)], idx_v)

        def gbody(k, carry2):
            ids = idx_v[pl.ds(k * 16, 16)]
            q_v[pl.ds(k * 16, 16)] = plsc.load_gather(rank_v, [ids])
            return carry2

        lax.fori_loop(0, 8, gbody, 0)
        pltpu.async_copy(ge_hbm.at[q_v], rows_v, sem).wait()
        pltpu.sync_copy(rows_v, out_hbm.at[pl.ds(base, 128)])
        return carry

    lax.fori_loop(0, 2, body, 0)


def _tc1_body(hist_ref, x_ref, w1_ref, p1_ref, dis_ref):
    deg = jnp.sum(hist_ref[...].T, axis=1, keepdims=True)
    dis = lax.rsqrt(deg + 1.0)
    dis_ref[...] = dis
    p1_ref[...] = _dot(x_ref[...], w1_ref[...]) * dis


def _tc2_body(acc_ref, p1_ref, dis_ref, b1_ref, w2_ref, p2_ref):
    dis = dis_ref[...]
    acc = jnp.concatenate([acc_ref[0], acc_ref[1]], axis=1)
    h1 = jnp.maximum(dis * (acc + p1_ref[...]) + b1_ref[...], 0.0)
    p2_ref[...] = _dot(h1, w2_ref[...]) * dis


def _tc3_body(acc_ref, p2_ref, dis_ref, b2_ref, h2_ref):
    dis = dis_ref[...]
    acc = jnp.concatenate([acc_ref[0], acc_ref[1]], axis=1)
    h2 = jnp.maximum(dis * (acc + p2_ref[...]) + b2_ref[...], 0.0)
    rid = lax.broadcasted_iota(jnp.int32, (NPAD, 1), 0)
    h2_ref[...] = jnp.where(rid < N_NODES, h2, 0.0)


def _tc4_body(sums_ref, cnt_ref, pres_ref, ge_ref, rank_ref):
    sums = sums_ref[0] + sums_ref[1]
    cnt = jnp.sum(cnt_ref[...].T, axis=1, keepdims=True)
    ge_ref[...] = sums / jnp.maximum(cnt, 1.0)
    pres_cnt = jnp.sum(pres_ref[...].T, axis=1, keepdims=True)
    pres = jnp.where(pres_cnt > 0.0, 1.0, 0.0)
    row = lax.broadcasted_iota(jnp.int32, (GPAD, GPAD), 0)
    col = lax.broadcasted_iota(jnp.int32, (GPAD, GPAD), 1)
    tri = jnp.where(row > col, 1.0, 0.0)
    rank_ref[...] = _dot(tri, pres).astype(jnp.int32)


def _tc5_body(fe_ref, te_ref, wr1_ref, br1_ref, wr2_ref, out_ref):
    cat = jnp.concatenate([fe_ref[...], te_ref[...]], axis=1)
    h = jnp.maximum(_dot(cat, wr1_ref[...]) + br1_ref[...], 0.0)
    out_ref[...] = _dot(h, wr2_ref[...])


def kernel(x, edge_index, batch, drug_drug_batch, W1, b1, W2, b2, Wr1, br1, Wr2, br2):
    f32 = jnp.float32
    pad_i = jnp.full((EPAD - N_EDGES,), NPAD - 1, jnp.int32)
    srcp = jnp.concatenate([edge_index[0], pad_i])
    dstp = jnp.concatenate([edge_index[1], pad_i])
    srcp2 = srcp.reshape(EPAD // 128, 128)
    dstp2 = dstp.reshape(EPAD // 128, 128)
    ddb_flat = drug_drug_batch.reshape(-1)

    z_nodes64 = jnp.zeros((NPAD, HCH), f32)
    z_1d = jnp.zeros((NPAD,), f32)
    z_g = jnp.zeros((GPAD, CH), f32)
    xp = jnp.pad(x, ((0, NPAD - N_NODES), (0, 0)))
    srchi2 = srcp2 + NPAD

    hist = _sc_hist()(dstp, z_1d)

    p1, dis = pl.pallas_call(
        _tc1_body,
        out_shape=[
            jax.ShapeDtypeStruct((NPAD, CH), f32),
            jax.ShapeDtypeStruct((NPAD, 1), f32),
        ],
    )(hist, xp, W1)

    t1 = jnp.concatenate([p1[:, :HCH], p1[:, HCH:]], axis=0)
    acc1 = _sc_edge_pass()(t1, srcp2, srchi2, dstp2, z_nodes64)

    p2 = pl.pallas_call(
        _tc2_body,
        out_shape=jax.ShapeDtypeStruct((NPAD, CH), f32),
    )(acc1, p1, dis, b1.reshape(1, CH), W2)

    t2 = jnp.concatenate([p2[:, :HCH], p2[:, HCH:]], axis=0)
    acc2 = _sc_edge_pass()(t2, srcp2, srchi2, dstp2, z_nodes64)

    h2 = pl.pallas_call(
        _tc3_body,
        out_shape=jax.ShapeDtypeStruct((NPAD, CH), f32),
    )(acc2, p2, dis, b2.reshape(1, CH))

    batchp = jnp.pad(batch, (0, NPAD - N_NODES))
    batchp2 = batchp.reshape(256, 40)
    z_1g = jnp.zeros((GPAD,), f32)
    sums_p, cnt_p, pres_p = _sc_pool()(h2, batchp, batchp2, ddb_flat, z_g, z_1g)

    ge, rank2d = pl.pallas_call(
        _tc4_body,
        out_shape=[
            jax.ShapeDtypeStruct((GPAD, CH), f32),
            jax.ShapeDtypeStruct((GPAD, 1), jnp.int32),
        ],
    )(sums_p, cnt_p, pres_p)
    rank = rank2d.reshape(GPAD)

    rows = _sc_pair_gather()(rank, ddb_flat, ge)
    fe = rows[:N_PAIRS]
    te = rows[N_PAIRS:]

    out = pl.pallas_call(
        _tc5_body,
        out_shape=jax.ShapeDtypeStruct((N_PAIRS, 1), f32),
    )(fe, te, Wr1, br1.reshape(1, -1), Wr2)
    return out


# speedup vs baseline: 1.8077x; 1.8077x over previous
import dataclasses
import functools

import jax
import jax.numpy as jnp
from jax import lax
from jax.experimental import pallas as pl
from jax.experimental.pallas import tpu as pltpu
from jax.experimental.pallas import tpu_sc as plsc

N_NODES = 10000
N_EDGES = 320000
EPAD = 327680
N_GRAPHS = 1000
GPAD = 1024
NPAD = 10240
CH = 128
N_PAIRS = 4096

_HIGH = jax.lax.Precision.HIGHEST


@functools.cache
def _mesh():
    return plsc.VectorSubcoreMesh(core_axis_name="c", subcore_axis_name="s")


def _sc_compiler_params():
    cp = pltpu.CompilerParams()
    if "needs_layout_passes" in pltpu.CompilerParams.__dataclass_fields__:
        cp = dataclasses.replace(cp, needs_layout_passes=False)
    return cp


def _sc_untiled_params():
    cp = pltpu.CompilerParams()
    if "use_tc_tiling_on_sc" in pltpu.CompilerParams.__dataclass_fields__:
        cp = dataclasses.replace(cp, use_tc_tiling_on_sc=False)
    return cp


def _dot(a, b):
    return jnp.dot(a, b, preferred_element_type=jnp.float32)


@functools.cache
def _sc_hist():
    return pl.kernel(
        _sc_hist_body,
        out_type=jax.ShapeDtypeStruct((32, NPAD), jnp.float32),
        mesh=_mesh(),
        scratch_types=[
            pltpu.VMEM((EPAD // 32,), jnp.int32),
            pltpu.VMEM((NPAD,), jnp.float32),
        ],
        compiler_params=_sc_compiler_params(),
    )


def _sc_hist_body(dst_hbm, z1d_hbm, out_hbm, idx_v, hist_v):
    cid = lax.axis_index("c")
    sid = lax.axis_index("s")
    w = cid * 16 + sid
    pltpu.sync_copy(z1d_hbm, hist_v)
    pltpu.sync_copy(dst_hbm.at[pl.ds(w * (EPAD // 32), EPAD // 32)], idx_v)
    ones16 = jnp.ones((16,), jnp.float32)

    def body(i, carry):
        for k in range(16):
            ids = idx_v[pl.ds(i * 256 + k * 16, 16)]
            plsc.addupdate_scatter(hist_v, [ids], ones16)
        return carry

    lax.fori_loop(0, EPAD // 32 // 256, body, 0)
    pltpu.sync_copy(hist_v, out_hbm.at[w])


@functools.cache
def _sc_edge_pass():
    return pl.kernel(
        _sc_edge_pass_body,
        out_type=jax.ShapeDtypeStruct((2, NPAD, CH), jnp.float32),
        mesh=_mesh(),
        scratch_types=[
            pltpu.VMEM((2, 128), jnp.int32),
            pltpu.VMEM((2, 128), jnp.int32),
            pltpu.VMEM((2, 128, CH), jnp.float32),
            pltpu.VMEM_SHARED((NPAD, CH), jnp.float32),
            pltpu.SemaphoreType.DMA((2,)),
        ],
    )


def _sc_edge_pass_body(table_hbm, src_hbm, dst_hbm, zeros_hbm, out_hbm,
                       src_v, dst_v, rows_v, acc_sh, sem):
    cid = lax.axis_index("c")
    sid = lax.axis_index("s")
    w = cid * 16 + sid
    r0 = sid * (NPAD // 16)
    pltpu.sync_copy(zeros_hbm.at[pl.ds(r0, NPAD // 16)], acc_sh.at[pl.ds(r0, NPAD // 16)])
    plsc.subcore_barrier()
    n = 78 + jnp.where(w < 4, 1, 0)

    def fetch(i, b):
        ofs = (w + i * 32) * 128
        pltpu.sync_copy(src_hbm.at[pl.ds(ofs, 128)], src_v.at[b])
        pltpu.sync_copy(dst_hbm.at[pl.ds(ofs, 128)], dst_v.at[b])
        pltpu.async_copy(table_hbm.at[src_v.at[b]], rows_v.at[b], sem.at[b])

    fetch(0, 0)

    def body(i, carry):
        b = lax.rem(i, 2)
        bn = lax.rem(i + 1, 2)

        @pl.when(i + 1 < n)
        def _():
            fetch(i + 1, bn)

        pltpu.make_async_copy(table_hbm.at[src_v.at[b]], rows_v.at[b], sem.at[b]).wait()
        pltpu.sync_copy(rows_v.at[b], acc_sh.at[dst_v.at[b]], add=True)
        return carry

    lax.fori_loop(0, n, body, 0)
    plsc.subcore_barrier()
    pltpu.sync_copy(acc_sh.at[pl.ds(r0, NPAD // 16)], out_hbm.at[cid, pl.ds(r0, NPAD // 16)])


@functools.cache
def _sc_pool():
    return pl.kernel(
        _sc_pool_body,
        out_type=[
            jax.ShapeDtypeStruct((2, GPAD, CH), jnp.float32),
            jax.ShapeDtypeStruct((32, GPAD), jnp.float32),
            jax.ShapeDtypeStruct((32, GPAD), jnp.float32),
        ],
        mesh=_mesh(),
        scratch_types=[
            pltpu.VMEM((NPAD // 32,), jnp.int32),
            pltpu.VMEM((8, 40), jnp.int32),
            pltpu.VMEM((256,), jnp.int32),
            pltpu.VMEM((NPAD // 32, CH), jnp.float32),
            pltpu.VMEM((GPAD,), jnp.float32),
            pltpu.VMEM((GPAD,), jnp.float32),
            pltpu.VMEM_SHARED((GPAD, CH), jnp.float32),
        ],
        compiler_params=_sc_compiler_params(),
    )


def _sc_pool_body(h2_hbm, batch_hbm, batch2_hbm, ddb_hbm, zrow_hbm, z1g_hbm,
                  sums_o, cnt_o, pres_o,
                  bidx_v, bidx2_v, didx_v, rows_v, cnt_v, pres_v,
                  sums_sh):
    cid = lax.axis_index("c")
    sid = lax.axis_index("s")
    w = cid * 16 + sid
    g0 = sid * (GPAD // 16)
    gs = GPAD // 16
    R = NPAD // 32
    pltpu.sync_copy(zrow_hbm.at[pl.ds(g0, gs)], sums_sh.at[pl.ds(g0, gs)])
    pltpu.sync_copy(z1g_hbm, cnt_v)
    pltpu.sync_copy(z1g_hbm, pres_v)
    pltpu.sync_copy(batch_hbm.at[pl.ds(w * R, R)], bidx_v)
    pltpu.sync_copy(batch2_hbm.at[pl.ds(w * 8, 8)], bidx2_v)
    pltpu.sync_copy(ddb_hbm.at[pl.ds(w * 256, 256)], didx_v)
    pltpu.sync_copy(h2_hbm.at[pl.ds(w * R, R)], rows_v)
    plsc.subcore_barrier()

    def nbody(c, carry):
        pltpu.sync_copy(rows_v.at[pl.ds(c * 40, 40)], sums_sh.at[bidx2_v.at[c]], add=True)
        return carry

    lax.fori_loop(0, 8, nbody, 0)

    ones16 = jnp.ones((16,), jnp.float32)
    base = w * R

    def cbody(k, carry):
        ids = bidx_v[pl.ds(k * 16, 16)]
        gidx = base + k * 16 + lax.iota(jnp.int32, 16)
        vals = jnp.where(gidx < N_NODES, 1.0, 0.0)
        plsc.addupdate_scatter(cnt_v, [ids], vals)
        return carry

    lax.fori_loop(0, R // 16, cbody, 0)

    def pbody(c, carry):
        for k in range(8):
            ids = didx_v[pl.ds(c * 128 + k * 16, 16)]
            plsc.addupdate_scatter(pres_v, [ids], ones16)
        return carry

    lax.fori_loop(0, 2, pbody, 0)
    plsc.subcore_barrier()
    pltpu.sync_copy(sums_sh.at[pl.ds(g0, gs)], sums_o.at[cid, pl.ds(g0, gs)])
    pltpu.sync_copy(cnt_v, cnt_o.at[w])
    pltpu.sync_copy(pres_v, pres_o.at[w])


@functools.cache
def _sc_pair_gather():
    return pl.kernel(
        _sc_pair_gather_body,
        out_type=jax.ShapeDtypeStruct((2 * N_PAIRS, CH), jnp.float32),
        mesh=_mesh(),
        scratch_types=[
            pltpu.VMEM((GPAD,), jnp.int32),
            pltpu.VMEM((128,), jnp.int32),
            pltpu.VMEM((128,), jnp.int32),
            pltpu.VMEM((128, CH), jnp.float32),
            pltpu.SemaphoreType.DMA,
        ],
        compiler_params=_sc_compiler_params(),
    )


def _sc_pair_gather_body(rank_hbm, ddb_hbm, ge_hbm, out_hbm,
                         rank_v, idx_v, q_v, rows_v, sem):
    cid = lax.axis_index("c")
    sid = lax.axis_index("s")
    w = cid * 16 + sid
    pltpu.sync_copy(rank_hbm, rank_v)

    def body(c, carry):
        base = w * 256 + c * 128
        pltpu.sync_copy(ddb_hbm.at[pl.ds(base, 128)], idx_v)

        def gbody(k, carry2):
            ids = idx_v[pl.ds(k * 16, 16)]
            q_v[pl.ds(k * 16, 16)] = plsc.load_gather(rank_v, [ids])
            return carry2

        lax.fori_loop(0, 8, gbody, 0)
        pltpu.async_copy(ge_hbm.at[q_v], rows_v, sem).wait()
        pltpu.sync_copy(rows_v, out_hbm.at[pl.ds(base, 128)])
        return carry

    lax.fori_loop(0, 2, body, 0)


def _tc1_body(hist_ref, x_ref, w1_ref, p1_ref, dis_ref):
    deg = jnp.sum(hist_ref[...].T, axis=1, keepdims=True)
    dis = lax.rsqrt(deg + 1.0)
    dis_ref[...] = dis
    p1_ref[...] = _dot(x_ref[...], w1_ref[...]) * dis


def _tc2_body(acc_ref, p1_ref, dis_ref, b1_ref, w2_ref, p2_ref):
    dis = dis_ref[...]
    h1 = jnp.maximum(dis * (acc_ref[0] + acc_ref[1] + p1_ref[...]) + b1_ref[...], 0.0)
    p2_ref[...] = _dot(h1, w2_ref[...]) * dis


def _tc3_body(acc_ref, p2_ref, dis_ref, b2_ref, h2_ref):
    dis = dis_ref[...]
    h2 = jnp.maximum(dis * (acc_ref[0] + acc_ref[1] + p2_ref[...]) + b2_ref[...], 0.0)
    rid = lax.broadcasted_iota(jnp.int32, (NPAD, 1), 0)
    h2_ref[...] = jnp.where(rid < N_NODES, h2, 0.0)


def _tc4_body(sums_ref, cnt_ref, pres_ref, ge_ref, rank_ref):
    sums = sums_ref[0] + sums_ref[1]
    cnt = jnp.sum(cnt_ref[...].T, axis=1, keepdims=True)
    ge_ref[...] = sums / jnp.maximum(cnt, 1.0)
    pres_cnt = jnp.sum(pres_ref[...].T, axis=1, keepdims=True)
    pres = jnp.where(pres_cnt > 0.0, 1.0, 0.0)
    row = lax.broadcasted_iota(jnp.int32, (GPAD, GPAD), 0)
    col = lax.broadcasted_iota(jnp.int32, (GPAD, GPAD), 1)
    tri = jnp.where(row > col, 1.0, 0.0)
    rank_ref[...] = _dot(tri, pres).astype(jnp.int32)


def _tc5_body(fe_ref, te_ref, wr1_ref, br1_ref, wr2_ref, out_ref):
    cat = jnp.concatenate([fe_ref[...], te_ref[...]], axis=1)
    h = jnp.maximum(_dot(cat, wr1_ref[...]) + br1_ref[...], 0.0)
    out_ref[...] = _dot(h, wr2_ref[...])


def kernel(x, edge_index, batch, drug_drug_batch, W1, b1, W2, b2, Wr1, br1, Wr2, br2):
    f32 = jnp.float32
    pad_i = jnp.full((EPAD - N_EDGES,), NPAD - 1, jnp.int32)
    srcp = jnp.concatenate([edge_index[0], pad_i])
    dstp = jnp.concatenate([edge_index[1], pad_i])
    srcp2 = srcp.reshape(EPAD // 128, 128)
    dstp2 = dstp.reshape(EPAD // 128, 128)
    ddb_flat = drug_drug_batch.reshape(-1)

    z_nodes = jnp.zeros((NPAD, CH), f32)
    z_1d = jnp.zeros((NPAD,), f32)
    z_g = jnp.zeros((GPAD, CH), f32)
    xp = jnp.pad(x, ((0, NPAD - N_NODES), (0, 0)))

    hist = _sc_hist()(dstp, z_1d)

    p1, dis = pl.pallas_call(
        _tc1_body,
        out_shape=[
            jax.ShapeDtypeStruct((NPAD, CH), f32),
            jax.ShapeDtypeStruct((NPAD, 1), f32),
        ],
    )(hist, xp, W1)

    acc1 = _sc_edge_pass()(p1, srcp, dstp, z_nodes)

    p2 = pl.pallas_call(
        _tc2_body,
        out_shape=jax.ShapeDtypeStruct((NPAD, CH), f32),
    )(acc1, p1, dis, b1.reshape(1, CH), W2)

    acc2 = _sc_edge_pass()(p2, srcp, dstp, z_nodes)

    h2 = pl.pallas_call(
        _tc3_body,
        out_shape=jax.ShapeDtypeStruct((NPAD, CH), f32),
    )(acc2, p2, dis, b2.reshape(1, CH))

    batchp = jnp.pad(batch, (0, NPAD - N_NODES))
    batchp2 = batchp.reshape(256, 40)
    z_1g = jnp.zeros((GPAD,), f32)
    sums_p, cnt_p, pres_p = _sc_pool()(h2, batchp, batchp2, ddb_flat, z_g, z_1g)

    ge, rank2d = pl.pallas_call(
        _tc4_body,
        out_shape=[
            jax.ShapeDtypeStruct((GPAD, CH), f32),
            jax.ShapeDtypeStruct((GPAD, 1), jnp.int32),
        ],
    )(sums_p, cnt_p, pres_p)
    rank = rank2d.reshape(GPAD)

    rows = _sc_pair_gather()(rank, ddb_flat, ge)
    fe = rows[:N_PAIRS]
    te = rows[N_PAIRS:]

    out = pl.pallas_call(
        _tc5_body,
        out_shape=jax.ShapeDtypeStruct((N_PAIRS, 1), f32),
    )(fe, te, Wr1, br1.reshape(1, -1), Wr2)
    return out
